# Initial kernel scaffold; baseline (speedup 1.0000x reference)
import jax
import jax.numpy as jnp
from jax.experimental import pallas as pl


def kernel(feature_bank, ind, feature):
    raise NotImplementedError("write your pallas kernel here")


# speedup vs baseline: 1.0784x; 1.0784x over previous
import functools

import jax
import jax.numpy as jnp
from jax import lax
from jax.experimental import pallas as pl
from jax.experimental.pallas import tpu as pltpu
from jax.experimental.pallas import tpu_sc as plsc

LENGTH = 100000
FEAT_DIM = 128
BATCH = 16384

NC = 2
NS = 16
NW = NC * NS
B_PER_W = BATCH // NW
SUB = 64
NSUB = B_PER_W // SUB
QCH = 4096
NQ = BATCH // QCH
LANE = 16

_MESH = plsc.VectorSubcoreMesh(core_axis_name="c", subcore_axis_name="s")


def _vrsqrt(sv):
    sc = jnp.maximum(sv, 1e-24)
    i = lax.bitcast_convert_type(sc, jnp.int32)
    i = 0x5F3759DF - lax.shift_right_logical(i, 1)
    y = lax.bitcast_convert_type(i, jnp.float32)
    for _ in range(3):
        y = y * (1.5 - 0.5 * sc * y * y)
    return y


def _sc_body(bank_hbm, ind_hbm, fnorm_hbm, out_hbm,
             w_ref, idxq, ibuf, jbuf, ivs, oldv, fnv, jsp, sem):
    cid = lax.axis_index("c")
    sid = lax.axis_index("s")
    lanes = lax.iota(jnp.int32, LANE)

    @pl.when(sid == 0)
    def _phase_a():
        def q_loop(q, carry):
            pltpu.sync_copy(ind_hbm.at[pl.ds(q * QCH, QCH)], idxq)

            def c_loop2(ci, c2):
                base_i = ci * LANE
                v = idxq[pl.ds(base_i, LANE)]
                dup = lanes < 0
                for s in range(1, LANE):
                    shifted = plsc.load_gather(
                        idxq, [base_i + jnp.minimum(lanes + s, LANE - 1)])
                    valid = (lanes + s) < LANE
                    dup = jnp.logical_or(
                        dup, jnp.logical_and(valid, v == shifted))
                keep = jnp.logical_not(dup)
                gpos = q * QCH + base_i + lanes
                plsc.store_scatter(w_ref, [v], gpos, mask=keep)
                return c2

            return lax.fori_loop(0, QCH // LANE, c_loop2, carry)

        lax.fori_loop(0, NQ, q_loop, 0)

        def q2_loop(q, carry):
            pltpu.sync_copy(ind_hbm.at[pl.ds(q * QCH, QCH)], idxq)

            def c2_loop(ci, c2):
                v = idxq[pl.ds(ci * LANE, LANE)]
                w = plsc.load_gather(w_ref, [v])
                idxq[pl.ds(ci * LANE, LANE)] = w
                return c2

            lax.fori_loop(0, QCH // LANE, c2_loop, 0)
            pltpu.sync_copy(idxq, jsp.at[pl.ds(q * QCH, QCH)])
            return carry

        lax.fori_loop(0, NQ, q2_loop, 0)

    plsc.subcore_barrier()

    wid = sid * NC + cid
    base = wid * B_PER_W
    pltpu.sync_copy(ind_hbm.at[pl.ds(base, B_PER_W)], ibuf)
    pltpu.sync_copy(jsp.at[pl.ds(base, B_PER_W)], jbuf)

    for s in range(NSUB):
        for k in range(SUB // LANE):
            ivs[s, pl.ds(k * LANE, LANE)] = ibuf[pl.ds(s * SUB + k * LANE, LANE)]

    for s in range(NSUB):
        pltpu.async_copy(
            bank_hbm.at[ibuf.at[pl.ds(s * SUB, SUB)]], oldv, sem).wait()
        pltpu.async_copy(
            fnorm_hbm.at[jbuf.at[pl.ds(s * SUB, SUB)]], fnv, sem).wait()

        def row_body(r, carry):
            u = [oldv[r, pl.ds(16 * k, 16)] + fnv[r, pl.ds(16 * k, 16)]
                 for k in range(8)]
            ss = u[0] * u[0]
            for k in range(1, 8):
                ss = ss + u[k] * u[k]
            y = _vrsqrt(jnp.broadcast_to(jnp.sum(ss), (LANE,)))
            for k in range(8):
                fnv[r, pl.ds(16 * k, 16)] = u[k] * y
            return carry

        lax.fori_loop(0, SUB, row_body, 0)
        pltpu.async_copy(fnv, out_hbm.at[ivs.at[s]], sem).wait()


_sc_update = pl.kernel(
    _sc_body,
    out_type=(),
    mesh=_MESH,
    compiler_params=pltpu.CompilerParams(needs_layout_passes=False),
    scratch_types=[
        pltpu.VMEM((LENGTH,), jnp.int32),
        pltpu.VMEM((QCH,), jnp.int32),
        pltpu.VMEM((B_PER_W,), jnp.int32),
        pltpu.VMEM((B_PER_W,), jnp.int32),
        pltpu.VMEM((NSUB, SUB), jnp.int32),
        pltpu.VMEM((SUB, FEAT_DIM), jnp.float32),
        pltpu.VMEM((SUB, FEAT_DIM), jnp.float32),
        pltpu.VMEM_SHARED((BATCH,), jnp.int32),
        pltpu.SemaphoreType.DMA,
    ],
)


def _fnorm_body(x_ref, o_ref):
    x = x_ref[...]
    ss = jnp.sum(x * x, axis=1, keepdims=True)
    o_ref[...] = x * lax.rsqrt(jnp.maximum(ss, 1e-24))


_FN_BLOCK = 2048

_fnorm_call = pl.pallas_call(
    _fnorm_body,
    out_shape=jax.ShapeDtypeStruct((BATCH, FEAT_DIM), jnp.float32),
    grid=(BATCH // _FN_BLOCK,),
    in_specs=[pl.BlockSpec((_FN_BLOCK, FEAT_DIM), lambda i: (i, 0))],
    out_specs=pl.BlockSpec((_FN_BLOCK, FEAT_DIM), lambda i: (i, 0)),
)


def kernel(feature_bank, ind, feature):
    ind32 = ind.astype(jnp.int32)
    fnorm = _fnorm_call(feature)
    out_ref = jax.new_ref(feature_bank)
    _sc_update(feature_bank, ind32, fnorm, out_ref)
    return out_ref[...]


# speedup vs baseline: 1.3223x; 1.2262x over previous
import jax
import jax.numpy as jnp
from jax import lax
from jax.experimental import pallas as pl
from jax.experimental.pallas import tpu as pltpu
from jax.experimental.pallas import tpu_sc as plsc

LENGTH = 100000
FEAT_DIM = 128
BATCH = 16384

NC = 2
NS = 16
NW = NC * NS
B_PER_W = BATCH // NW
SUB = 32
NSUB = B_PER_W // SUB
QCH = 2048
NQ = BATCH // QCH
LANE = 16

_MESH = plsc.VectorSubcoreMesh(core_axis_name="c", subcore_axis_name="s")


def _vrsqrt(sv):
    sc = jnp.maximum(sv, 1e-24)
    i = lax.bitcast_convert_type(sc, jnp.int32)
    i = 0x5F3759DF - lax.shift_right_logical(i, 1)
    y = lax.bitcast_convert_type(i, jnp.float32)
    for _ in range(3):
        y = y * (1.5 - 0.5 * sc * y * y)
    return y


def _sc_body(bank_hbm, ind_hbm, fnorm_hbm, out_hbm,
             w_ref, idxq, ibuf, jbuf, ivs,
             oldv0, oldv1, fnv0, fnv1, jsp,
             gsem0, gsem1, ssem0, ssem1):
    cid = lax.axis_index("c")
    sid = lax.axis_index("s")
    lanes = lax.iota(jnp.int32, LANE)

    @pl.when(sid == 0)
    def _phase_a():
        def q_loop(q, carry):
            pltpu.sync_copy(ind_hbm.at[pl.ds(q * QCH, QCH)], idxq)

            def c_loop(ci, c2):
                v = idxq[pl.ds(ci * LANE, LANE)]
                gpos = q * QCH + ci * LANE + lanes
                plsc.store_scatter(w_ref, [v], gpos)
                return c2

            return lax.fori_loop(0, QCH // LANE, c_loop, carry)

        lax.fori_loop(0, NQ, q_loop, 0)

        def q2_loop(q, carry):
            pltpu.sync_copy(ind_hbm.at[pl.ds(q * QCH, QCH)], idxq)

            def c2_loop(ci, c2):
                v = idxq[pl.ds(ci * LANE, LANE)]
                w = plsc.load_gather(w_ref, [v])
                idxq[pl.ds(ci * LANE, LANE)] = w
                return c2

            lax.fori_loop(0, QCH // LANE, c2_loop, 0)
            pltpu.sync_copy(idxq, jsp.at[pl.ds(q * QCH, QCH)])
            return carry

        lax.fori_loop(0, NQ, q2_loop, 0)

    plsc.subcore_barrier()

    wid = sid * NC + cid
    base = wid * B_PER_W
    pltpu.sync_copy(ind_hbm.at[pl.ds(base, B_PER_W)], ibuf)
    pltpu.sync_copy(jsp.at[pl.ds(base, B_PER_W)], jbuf)

    for s in range(NSUB):
        for k in range(SUB // LANE):
            ivs[s, pl.ds(k * LANE, LANE)] = ibuf[pl.ds(s * SUB + k * LANE, LANE)]

    oldv = [oldv0, oldv1]
    fnv = [fnv0, fnv1]
    gsem = [gsem0, gsem1]
    ssem = [ssem0, ssem1]

    def start_gathers(s, b):
        c1 = pltpu.async_copy(
            bank_hbm.at[ibuf.at[pl.ds(s * SUB, SUB)]], oldv[b], gsem[b])
        c2 = pltpu.async_copy(
            fnorm_hbm.at[jbuf.at[pl.ds(s * SUB, SUB)]], fnv[b], gsem[b])
        return (c1, c2)

    gd = [None, None]
    sd = [None, None]
    gd[0] = start_gathers(0, 0)
    for s in range(NSUB):
        b = s & 1
        nb = 1 - b
        if s + 1 < NSUB:
            if sd[nb] is not None:
                sd[nb].wait()
                sd[nb] = None
            gd[nb] = start_gathers(s + 1, nb)
        gd[b][0].wait()
        gd[b][1].wait()

        ob, fb = oldv[b], fnv[b]

        def row_body(r, carry):
            u = [ob[r, pl.ds(16 * k, 16)] + fb[r, pl.ds(16 * k, 16)]
                 for k in range(8)]
            ss = u[0] * u[0]
            for k in range(1, 8):
                ss = ss + u[k] * u[k]
            y = _vrsqrt(jnp.broadcast_to(jnp.sum(ss), (LANE,)))
            for k in range(8):
                fb[r, pl.ds(16 * k, 16)] = u[k] * y
            return carry

        lax.fori_loop(0, SUB, row_body, 0)
        sd[b] = pltpu.async_copy(fnv[b], out_hbm.at[ivs.at[s]], ssem[b])
    for b in range(2):
        if sd[b] is not None:
            sd[b].wait()


_sc_update = pl.kernel(
    _sc_body,
    out_type=(),
    mesh=_MESH,
    compiler_params=pltpu.CompilerParams(needs_layout_passes=False),
    scratch_types=[
        pltpu.VMEM((LENGTH,), jnp.int32),
        pltpu.VMEM((QCH,), jnp.int32),
        pltpu.VMEM((B_PER_W,), jnp.int32),
        pltpu.VMEM((B_PER_W,), jnp.int32),
        pltpu.VMEM((NSUB, SUB), jnp.int32),
        pltpu.VMEM((SUB, FEAT_DIM), jnp.float32),
        pltpu.VMEM((SUB, FEAT_DIM), jnp.float32),
        pltpu.VMEM((SUB, FEAT_DIM), jnp.float32),
        pltpu.VMEM((SUB, FEAT_DIM), jnp.float32),
        pltpu.VMEM_SHARED((BATCH,), jnp.int32),
        pltpu.SemaphoreType.DMA,
        pltpu.SemaphoreType.DMA,
        pltpu.SemaphoreType.DMA,
        pltpu.SemaphoreType.DMA,
    ],
)


def _fnorm_body(x_ref, o_ref):
    x = x_ref[...]
    ss = jnp.sum(x * x, axis=1, keepdims=True)
    o_ref[...] = x * lax.rsqrt(jnp.maximum(ss, 1e-24))


_FN_BLOCK = 2048

_fnorm_call = pl.pallas_call(
    _fnorm_body,
    out_shape=jax.ShapeDtypeStruct((BATCH, FEAT_DIM), jnp.float32),
    grid=(BATCH // _FN_BLOCK,),
    in_specs=[pl.BlockSpec((_FN_BLOCK, FEAT_DIM), lambda i: (i, 0))],
    out_specs=pl.BlockSpec((_FN_BLOCK, FEAT_DIM), lambda i: (i, 0)),
)


def kernel(feature_bank, ind, feature):
    ind32 = ind.astype(jnp.int32)
    fnorm = _fnorm_call(feature)
    out_ref = jax.new_ref(feature_bank)
    _sc_update(feature_bank, ind32, fnorm, out_ref)
    return out_ref[...]


# speedup vs baseline: 1.7201x; 1.3008x over previous
import jax
import jax.numpy as jnp
from jax import lax
from jax.experimental import pallas as pl
from jax.experimental.pallas import tpu as pltpu
from jax.experimental.pallas import tpu_sc as plsc

LENGTH = 100000
FEAT_DIM = 128
BATCH = 16384

NC = 2
NS = 16
NW = NC * NS
B_PER_W = BATCH // NW
SUB = 128
NSUB = B_PER_W // SUB
LANE = 16

_MESH = plsc.VectorSubcoreMesh(core_axis_name="c", subcore_axis_name="s")
_SC_PARAMS = pltpu.CompilerParams(needs_layout_passes=False)


def _vrsqrt(sv):
    sc = jnp.maximum(sv, 1e-24)
    i = lax.bitcast_convert_type(sc, jnp.int32)
    i = 0x5F3759DF - lax.shift_right_logical(i, 1)
    y = lax.bitcast_convert_type(i, jnp.float32)
    for _ in range(3):
        y = y * (1.5 - 0.5 * sc * y * y)
    return y


def _winners_body(ind_hbm, j_hbm, w_ref, idxb):
    cid = lax.axis_index("c")
    sid = lax.axis_index("s")
    lanes = lax.iota(jnp.int32, LANE)

    @pl.when((sid == 0) & (cid == 0))
    def _():
        pltpu.sync_copy(ind_hbm, idxb)

        def c_loop(ci, c2):
            v = idxb[pl.ds(ci * LANE, LANE)]
            plsc.store_scatter(w_ref, [v], ci * LANE + lanes)
            return c2

        lax.fori_loop(0, BATCH // LANE, c_loop, 0)

        def c2_loop(ci, c2):
            v = idxb[pl.ds(ci * LANE, LANE)]
            idxb[pl.ds(ci * LANE, LANE)] = plsc.load_gather(w_ref, [v])
            return c2

        lax.fori_loop(0, BATCH // LANE, c2_loop, 0)
        pltpu.sync_copy(idxb, j_hbm)


_sc_winners = pl.kernel(
    _winners_body,
    out_type=jax.ShapeDtypeStruct((BATCH,), jnp.int32),
    mesh=_MESH,
    compiler_params=_SC_PARAMS,
    scratch_types=[
        pltpu.VMEM((LENGTH,), jnp.int32),
        pltpu.VMEM((BATCH,), jnp.int32),
    ],
)


def _apply_body(bank_hbm, ind_hbm, fnorm_hbm, j_hbm, out_hbm,
                ibuf, jbuf, ivs,
                oldv0, oldv1, fnv0, fnv1,
                gsem0, gsem1, ssem0, ssem1):
    cid = lax.axis_index("c")
    sid = lax.axis_index("s")
    wid = sid * NC + cid
    base = wid * B_PER_W
    pltpu.sync_copy(ind_hbm.at[pl.ds(base, B_PER_W)], ibuf)
    pltpu.sync_copy(j_hbm.at[pl.ds(base, B_PER_W)], jbuf)

    for s in range(NSUB):
        for k in range(SUB // LANE):
            ivs[s, pl.ds(k * LANE, LANE)] = ibuf[pl.ds(s * SUB + k * LANE, LANE)]

    oldv = [oldv0, oldv1]
    fnv = [fnv0, fnv1]
    gsem = [gsem0, gsem1]
    ssem = [ssem0, ssem1]

    def start_gathers(s, b):
        c1 = pltpu.async_copy(
            bank_hbm.at[ibuf.at[pl.ds(s * SUB, SUB)]], oldv[b], gsem[b])
        c2 = pltpu.async_copy(
            fnorm_hbm.at[jbuf.at[pl.ds(s * SUB, SUB)]], fnv[b], gsem[b])
        return (c1, c2)

    gd = [None, None]
    sd = [None, None]
    gd[0] = start_gathers(0, 0)
    for s in range(NSUB):
        b = s & 1
        nb = 1 - b
        if s + 1 < NSUB:
            if sd[nb] is not None:
                sd[nb].wait()
                sd[nb] = None
            gd[nb] = start_gathers(s + 1, nb)
        gd[b][0].wait()
        gd[b][1].wait()

        ob, fb = oldv[b], fnv[b]

        def row_body(r, carry):
            u = [ob[r, pl.ds(16 * k, 16)] + fb[r, pl.ds(16 * k, 16)]
                 for k in range(8)]
            ss = u[0] * u[0]
            for k in range(1, 8):
                ss = ss + u[k] * u[k]
            y = _vrsqrt(jnp.broadcast_to(jnp.sum(ss), (LANE,)))
            for k in range(8):
                fb[r, pl.ds(16 * k, 16)] = u[k] * y
            return carry

        lax.fori_loop(0, SUB, row_body, 0)
        sd[b] = pltpu.async_copy(fnv[b], out_hbm.at[ivs.at[s]], ssem[b])
    for b in range(2):
        if sd[b] is not None:
            sd[b].wait()


_sc_apply = pl.kernel(
    _apply_body,
    out_type=(),
    mesh=_MESH,
    compiler_params=_SC_PARAMS,
    scratch_types=[
        pltpu.VMEM((B_PER_W,), jnp.int32),
        pltpu.VMEM((B_PER_W,), jnp.int32),
        pltpu.VMEM((NSUB, SUB), jnp.int32),
        pltpu.VMEM((SUB, FEAT_DIM), jnp.float32),
        pltpu.VMEM((SUB, FEAT_DIM), jnp.float32),
        pltpu.VMEM((SUB, FEAT_DIM), jnp.float32),
        pltpu.VMEM((SUB, FEAT_DIM), jnp.float32),
        pltpu.SemaphoreType.DMA,
        pltpu.SemaphoreType.DMA,
        pltpu.SemaphoreType.DMA,
        pltpu.SemaphoreType.DMA,
    ],
)


def _fnorm_body(x_ref, o_ref):
    x = x_ref[...]
    ss = jnp.sum(x * x, axis=1, keepdims=True)
    o_ref[...] = x * lax.rsqrt(jnp.maximum(ss, 1e-24))


_FN_BLOCK = 2048

_fnorm_call = pl.pallas_call(
    _fnorm_body,
    out_shape=jax.ShapeDtypeStruct((BATCH, FEAT_DIM), jnp.float32),
    grid=(BATCH // _FN_BLOCK,),
    in_specs=[pl.BlockSpec((_FN_BLOCK, FEAT_DIM), lambda i: (i, 0))],
    out_specs=pl.BlockSpec((_FN_BLOCK, FEAT_DIM), lambda i: (i, 0)),
)


def kernel(feature_bank, ind, feature):
    ind32 = ind.astype(jnp.int32)
    j = _sc_winners(ind32)
    fnorm = _fnorm_call(feature)
    out_ref = jax.new_ref(feature_bank)
    _sc_apply(feature_bank, ind32, fnorm, j, out_ref)
    return out_ref[...]


# speedup vs baseline: 2.0273x; 1.1786x over previous
import jax
import jax.numpy as jnp
from jax import lax
from jax.experimental import pallas as pl
from jax.experimental.pallas import tpu as pltpu
from jax.experimental.pallas import tpu_sc as plsc

LENGTH = 100000
FEAT_DIM = 128
BATCH = 16384

NC = 2
NS = 16
NW = NC * NS
B_PER_W = BATCH // NW
SUB = 128
NSUB = B_PER_W // SUB
LANE = 16

_MESH = plsc.VectorSubcoreMesh(core_axis_name="c", subcore_axis_name="s")
_SC_PARAMS = pltpu.CompilerParams(needs_layout_passes=False)


def _vrsqrt(sv):
    sc = jnp.maximum(sv, 1e-24)
    i = lax.bitcast_convert_type(sc, jnp.int32)
    i = 0x5F3759DF - lax.shift_right_logical(i, 1)
    y = lax.bitcast_convert_type(i, jnp.float32)
    for _ in range(3):
        y = y * (1.5 - 0.5 * sc * y * y)
    return y


def _row_normalize(ob, fb, n_rows):
    @plsc.parallel_loop(0, n_rows, unroll=2)
    def _rows(r):
        u = [ob[r, pl.ds(16 * k, 16)] + fb[r, pl.ds(16 * k, 16)]
             for k in range(8)]
        ss = u[0] * u[0]
        for k in range(1, 8):
            ss = ss + u[k] * u[k]
        y = _vrsqrt(jnp.broadcast_to(jnp.sum(ss), (LANE,)))
        for k in range(8):
            fb[r, pl.ds(16 * k, 16)] = u[k] * y


def _winners_body(ind_hbm, j_hbm, w_ref, idxb):
    cid = lax.axis_index("c")
    sid = lax.axis_index("s")
    lanes = lax.iota(jnp.int32, LANE)

    @pl.when((sid == 0) & (cid == 0))
    def _():
        pltpu.sync_copy(ind_hbm, idxb)

        def c_loop(ci, c2):
            v = idxb[pl.ds(ci * LANE, LANE)]
            plsc.store_scatter(w_ref, [v], ci * LANE + lanes)
            return c2

        lax.fori_loop(0, BATCH // LANE, c_loop, 0)

        def c2_loop(ci, c2):
            v = idxb[pl.ds(ci * LANE, LANE)]
            idxb[pl.ds(ci * LANE, LANE)] = plsc.load_gather(w_ref, [v])
            return c2

        lax.fori_loop(0, BATCH // LANE, c2_loop, 0)
        pltpu.sync_copy(idxb, j_hbm)


_sc_winners = pl.kernel(
    _winners_body,
    out_type=jax.ShapeDtypeStruct((BATCH,), jnp.int32),
    mesh=_MESH,
    compiler_params=_SC_PARAMS,
    scratch_types=[
        pltpu.VMEM((LENGTH,), jnp.int32),
        pltpu.VMEM((BATCH,), jnp.int32),
    ],
)


def _apply_body(bank_hbm, ind_hbm, fnorm_hbm, j_hbm, out_hbm,
                ibuf, jbuf, ivs,
                oldv0, oldv1, fnv0, fnv1,
                gsem0, gsem1, ssem0, ssem1):
    cid = lax.axis_index("c")
    sid = lax.axis_index("s")
    wid = sid * NC + cid
    base = wid * B_PER_W
    pltpu.sync_copy(ind_hbm.at[pl.ds(base, B_PER_W)], ibuf)
    pltpu.sync_copy(j_hbm.at[pl.ds(base, B_PER_W)], jbuf)

    for s in range(NSUB):
        for k in range(SUB // LANE):
            ivs[s, pl.ds(k * LANE, LANE)] = ibuf[pl.ds(s * SUB + k * LANE, LANE)]

    oldv = [oldv0, oldv1]
    fnv = [fnv0, fnv1]
    gsem = [gsem0, gsem1]
    ssem = [ssem0, ssem1]

    def start_gathers(s, b):
        c1 = pltpu.async_copy(
            bank_hbm.at[ibuf.at[pl.ds(s * SUB, SUB)]], oldv[b], gsem[b])
        c2 = pltpu.async_copy(
            fnorm_hbm.at[jbuf.at[pl.ds(s * SUB, SUB)]], fnv[b], gsem[b])
        return (c1, c2)

    gd = [None, None]
    sd = [None, None]
    gd[0] = start_gathers(0, 0)
    for s in range(NSUB):
        b = s & 1
        nb = 1 - b
        if s + 1 < NSUB:
            if sd[nb] is not None:
                sd[nb].wait()
                sd[nb] = None
            gd[nb] = start_gathers(s + 1, nb)
        gd[b][0].wait()
        gd[b][1].wait()

        _row_normalize(oldv[b], fnv[b], SUB)
        sd[b] = pltpu.async_copy(fnv[b], out_hbm.at[ivs.at[s]], ssem[b])
    for b in range(2):
        if sd[b] is not None:
            sd[b].wait()


_sc_apply = pl.kernel(
    _apply_body,
    out_type=(),
    mesh=_MESH,
    compiler_params=_SC_PARAMS,
    scratch_types=[
        pltpu.VMEM((B_PER_W,), jnp.int32),
        pltpu.VMEM((B_PER_W,), jnp.int32),
        pltpu.VMEM((NSUB, SUB), jnp.int32),
        pltpu.VMEM((SUB, FEAT_DIM), jnp.float32),
        pltpu.VMEM((SUB, FEAT_DIM), jnp.float32),
        pltpu.VMEM((SUB, FEAT_DIM), jnp.float32),
        pltpu.VMEM((SUB, FEAT_DIM), jnp.float32),
        pltpu.SemaphoreType.DMA,
        pltpu.SemaphoreType.DMA,
        pltpu.SemaphoreType.DMA,
        pltpu.SemaphoreType.DMA,
    ],
)


def _fnorm_body(x_ref, o_ref):
    x = x_ref[...]
    ss = jnp.sum(x * x, axis=1, keepdims=True)
    o_ref[...] = x * lax.rsqrt(jnp.maximum(ss, 1e-24))


_FN_BLOCK = 2048

_fnorm_call = pl.pallas_call(
    _fnorm_body,
    out_shape=jax.ShapeDtypeStruct((BATCH, FEAT_DIM), jnp.float32),
    grid=(BATCH // _FN_BLOCK,),
    in_specs=[pl.BlockSpec((_FN_BLOCK, FEAT_DIM), lambda i: (i, 0))],
    out_specs=pl.BlockSpec((_FN_BLOCK, FEAT_DIM), lambda i: (i, 0)),
)


def kernel(feature_bank, ind, feature):
    ind32 = ind.astype(jnp.int32)
    j = _sc_winners(ind32)
    out_ref = jax.new_ref(feature_bank)
    fnorm = _fnorm_call(feature)
    _sc_apply(feature_bank, ind32, fnorm, j, out_ref)
    return out_ref[...]
